# baseline (speedup 1.0000x reference)
import functools
import math

import jax
import jax.numpy as jnp
from jax import lax
from jax.experimental import pallas as pl
from jax.experimental.pallas import tpu as pltpu
from jax.experimental.pallas import tpu_sc as plsc

EMB = 1024
H = 16
HD = 64
E = 8
S = 512
T = 2048
TB = 256
PB = 256
MAXB = T // PB * (E // 2) + E
PMAX = MAXB * PB
NA = 2 * T
EMB2 = EMB // 2

_INV_SQRT2 = 1.0 / math.sqrt(2.0)


def _gelu(s):
    return 0.5 * s * (1.0 + lax.erf(s * _INV_SQRT2))


def _pack_row(y):
    bl = lax.bitcast_convert_type(y[:, :EMB2], jnp.int32)
    br = lax.bitcast_convert_type(y[:, EMB2:], jnp.int32)
    hi = (bl + 0x8000) & jnp.int32(-65536)
    lo = lax.shift_right_logical(br + 0x8000, 16)
    return hi | lo


def _unpack_row(w):
    yl = lax.bitcast_convert_type(w & jnp.int32(-65536), jnp.float32)
    yr = lax.bitcast_convert_type(lax.shift_left(w, 16), jnp.float32)
    return jnp.concatenate([yl, yr], axis=1)


def _router_proj_kernel(x_ref, wr_ref, wh_ref, q_ref, gts_ref, dest_ref,
                        meta_ref, aidx_ref):
    b = pl.program_id(0)
    xb = x_ref[...]
    logits = lax.dot_general(wr_ref[...], xb, (((1,), (1,)), ((), ())),
                             preferred_element_type=jnp.float32)
    e_iota = lax.broadcasted_iota(jnp.int32, (E, TB), 0)
    m1 = jnp.max(logits, axis=0, keepdims=True)
    a1 = jnp.min(jnp.where(logits == m1, e_iota, E), axis=0, keepdims=True)
    l2 = jnp.where(e_iota == a1, -jnp.inf, logits)
    m2 = jnp.max(l2, axis=0, keepdims=True)
    a2 = jnp.min(jnp.where(l2 == m2, e_iota, E), axis=0, keepdims=True)
    g1 = 0.5 * jax.nn.sigmoid(m1)
    g2 = 0.5 * jax.nn.sigmoid(m2)
    aidx_ref[:, :, pl.ds(b * TB, TB)] = (
        jnp.concatenate([a1, a2], axis=0).reshape(2, 1, TB))
    gts_ref[...] = jnp.concatenate([g1, g2], axis=0).reshape(2, 1, TB)
    q = lax.dot_general(xb, wh_ref[...], (((1,), (1,)), ((), ())),
                        preferred_element_type=jnp.float32)
    q_ref[...] = _pack_row(q)

    @pl.when(b == T // TB - 1)
    def _():
        _meta_body(aidx_ref, dest_ref, meta_ref)


def _router_proj(x2d, W_router, W_h):
    return pl.pallas_call(
        _router_proj_kernel,
        grid=(T // TB,),
        in_specs=[
            pl.BlockSpec((TB, EMB), lambda b: (b, 0)),
            pl.BlockSpec((E, EMB), lambda b: (0, 0)),
            pl.BlockSpec((EMB, EMB), lambda b: (0, 0)),
        ],
        out_specs=[
            pl.BlockSpec((TB, EMB2), lambda b: (b, 0)),
            pl.BlockSpec((2, 1, TB), lambda b: (0, 0, b)),
            pl.BlockSpec((2, 1, T), lambda b: (0, 0, 0)),
            pl.BlockSpec((2, 128), lambda b: (0, 0)),
        ],
        out_shape=[
            jax.ShapeDtypeStruct((T, EMB2), jnp.int32),
            jax.ShapeDtypeStruct((2, 1, T), jnp.float32),
            jax.ShapeDtypeStruct((2, 1, T), jnp.int32),
            jax.ShapeDtypeStruct((2, 128), jnp.int32),
        ],
        scratch_shapes=[pltpu.VMEM((2, 1, T), jnp.int32)],
    )(x2d, W_router, W_h)


def _meta_body(aidx_ref, dest_ref, meta_ref):
    a1 = aidx_ref[0]
    a2 = aidx_ref[1]
    e_iota = lax.broadcasted_iota(jnp.int32, (E, T), 0)
    occ1 = (e_iota == a1).astype(jnp.int32)
    occ2 = (e_iota == a2).astype(jnp.int32)
    occ = occ1 + occ2
    c = occ
    sh = 1
    while sh < T:
        c = c + jnp.concatenate([jnp.zeros((E, sh), jnp.int32), c[:, :T - sh]], axis=1)
        sh *= 2
    rank = c - occ
    count = c[:, T - 1:T]
    pc = ((count + PB - 1) // PB) * PB
    o = pc
    sh = 1
    while sh < E:
        o = o + jnp.concatenate([jnp.zeros((sh, 1), jnp.int32), o[:E - sh]], axis=0)
        sh *= 2
    excl = o - pc
    nb_e = pc // PB
    blk_off = excl // PB
    nb_total = jnp.sum(nb_e, axis=0, keepdims=True)
    e_col = lax.broadcasted_iota(jnp.int32, (E, 128), 0)
    b_iota = lax.broadcasted_iota(jnp.int32, (E, 128), 1)
    belong = (b_iota >= blk_off) & (b_iota < blk_off + nb_e)
    be = jnp.sum(jnp.where(belong, e_col, 0), axis=0, keepdims=True)
    last = nb_total - 1
    bl = (blk_off <= last) & (last < blk_off + nb_e)
    e_last = jnp.sum(jnp.where(bl, e_col[:, :1], 0), axis=0, keepdims=True)
    b_row = lax.broadcasted_iota(jnp.int32, (1, 128), 1)
    be_final = jnp.where(b_row < nb_total, be, e_last)
    meta_ref[...] = jnp.concatenate(
        [be_final, jnp.broadcast_to(nb_total, (1, 128))], axis=0)
    slot = excl + rank
    d1 = jnp.sum(occ1 * slot, axis=0, keepdims=True)
    d2 = jnp.sum(occ2 * slot, axis=0, keepdims=True)
    dest_ref[...] = jnp.concatenate([d1, d2], axis=0).reshape(2, 1, T)


_SC_CHUNK = 128


def _sc_mesh():
    return plsc.VectorSubcoreMesh(core_axis_name="c", subcore_axis_name="s")


def _sc_pack(dest_flat, tok_ids, qi):
    info = plsc.get_sparse_core_info()
    nw = info.num_cores * info.num_subcores
    per_w = NA // nw

    @functools.partial(
        pl.kernel,
        mesh=_sc_mesh(),
        out_type=jax.ShapeDtypeStruct((PMAX, EMB2), jnp.int32),
        scratch_types=[
            pltpu.VMEM((_SC_CHUNK,), jnp.int32),
            pltpu.VMEM((_SC_CHUNK,), jnp.int32),
            pltpu.VMEM((_SC_CHUNK, EMB2), jnp.int32),
            pltpu.SemaphoreType.DMA,
        ],
    )
    def k(dest_hbm, tok_hbm, q_hbm, pq_hbm, tidx_v, didx_v, rows_v, sem):
        wid = lax.axis_index("s") * info.num_cores + lax.axis_index("c")
        base = wid * per_w
        for c in range(per_w // _SC_CHUNK):
            off = base + c * _SC_CHUNK
            pltpu.sync_copy(tok_hbm.at[pl.ds(off, _SC_CHUNK)], tidx_v)
            pltpu.sync_copy(dest_hbm.at[pl.ds(off, _SC_CHUNK)], didx_v)
            pltpu.async_copy(q_hbm.at[tidx_v], rows_v, sem).wait()
            pltpu.async_copy(rows_v, pq_hbm.at[didx_v], sem).wait()

    return k(dest_flat, tok_ids, qi)


def _sc_unpack(dest_flat, hpi):
    info = plsc.get_sparse_core_info()
    nw = info.num_cores * info.num_subcores
    per_w = NA // nw

    @functools.partial(
        pl.kernel,
        mesh=_sc_mesh(),
        out_type=jax.ShapeDtypeStruct((NA, EMB2), jnp.int32),
        scratch_types=[
            pltpu.VMEM((_SC_CHUNK,), jnp.int32),
            pltpu.VMEM((_SC_CHUNK, EMB2), jnp.int32),
            pltpu.SemaphoreType.DMA,
        ],
    )
    def k(dest_hbm, hp_hbm, h01_hbm, didx_v, rows_v, sem):
        wid = lax.axis_index("s") * info.num_cores + lax.axis_index("c")
        base = wid * per_w
        for c in range(per_w // _SC_CHUNK):
            off = base + c * _SC_CHUNK
            pltpu.sync_copy(dest_hbm.at[pl.ds(off, _SC_CHUNK)], didx_v)
            pltpu.async_copy(hp_hbm.at[didx_v], rows_v, sem).wait()
            pltpu.sync_copy(rows_v, h01_hbm.at[pl.ds(off, _SC_CHUNK)])

    return k(dest_flat, hpi)


def _ffn_block_kernel(be_ref, nbt_ref, pq_ref, k_ref, v_ref, hp_ref,
                      k16_ref, v16_ref):
    b = pl.program_id(0)
    prev = be_ref[jnp.maximum(b - 1, 0)]

    @pl.when(b == 0)
    def _():
        k16_ref[...] = k_ref[...].astype(jnp.bfloat16)
        v16_ref[...] = v_ref[...].astype(jnp.bfloat16)

    @pl.when((b > 0) & (be_ref[b] != prev) & (b < nbt_ref[0]))
    def _():
        k16_ref[...] = k_ref[...].astype(jnp.bfloat16)
        v16_ref[...] = v_ref[...].astype(jnp.bfloat16)

    @pl.when(b < nbt_ref[0])
    def _():
        qb = _unpack_row(pq_ref[...]).astype(jnp.bfloat16)
        hcols = []
        for h in range(H):
            qh = qb[:, h * HD:(h + 1) * HD]
            s = lax.dot_general(qh, k16_ref[h, 0], (((1,), (1,)), ((), ())),
                                preferred_element_type=jnp.float32,
                                precision=lax.Precision.DEFAULT)
            a = (s + s * lax.erf(s * _INV_SQRT2)).astype(jnp.bfloat16)
            hcols.append(lax.dot_general(a, v16_ref[h, 0], (((1,), (0,)), ((), ())),
                                         preferred_element_type=jnp.float32,
                                         precision=lax.Precision.DEFAULT))
        hp_ref[...] = _pack_row(jnp.concatenate(hcols, axis=1))


def _ffn_block(be_arr, nbt_arr, packed_q, k_ffwd, v_ffwd):
    grid_spec = pltpu.PrefetchScalarGridSpec(
        num_scalar_prefetch=2,
        grid=(MAXB,),
        in_specs=[
            pl.BlockSpec((PB, EMB2), lambda b, be, nbt: (b, 0)),
            pl.BlockSpec((H, 1, S, HD), lambda b, be, nbt: (0, be[b], 0, 0)),
            pl.BlockSpec((H, 1, S, HD), lambda b, be, nbt: (0, be[b], 0, 0)),
        ],
        out_specs=pl.BlockSpec((PB, EMB2), lambda b, be, nbt: (b, 0)),
        scratch_shapes=[
            pltpu.VMEM((H, 1, S, HD), jnp.bfloat16),
            pltpu.VMEM((H, 1, S, HD), jnp.bfloat16),
        ],
    )
    return pl.pallas_call(
        _ffn_block_kernel,
        grid_spec=grid_spec,
        out_shape=jax.ShapeDtypeStruct((PMAX, EMB2), jnp.int32),
    )(be_arr, nbt_arr, packed_q, k_ffwd, v_ffwd)


def _combine_kernel(h0_ref, h1_ref, gts_ref, wg_ref, o_ref):
    g0 = gts_ref[0, 0, :].reshape(TB, 1)
    g1 = gts_ref[1, 0, :].reshape(TB, 1)
    mixed = g0 * _unpack_row(h0_ref[...]) + g1 * _unpack_row(h1_ref[...])
    o_ref[...] = lax.dot_general(mixed, wg_ref[...], (((1,), (1,)), ((), ())),
                                 preferred_element_type=jnp.float32)


def _combine(h01, gts, W_g):
    nblk = T // TB
    return pl.pallas_call(
        _combine_kernel,
        grid=(nblk,),
        in_specs=[
            pl.BlockSpec((TB, EMB2), lambda b: (b, 0)),
            pl.BlockSpec((TB, EMB2), lambda b: (b + nblk, 0)),
            pl.BlockSpec((2, 1, TB), lambda b: (0, 0, b)),
            pl.BlockSpec((EMB, EMB), lambda b: (0, 0)),
        ],
        out_specs=pl.BlockSpec((TB, EMB), lambda b: (b, 0)),
        out_shape=jax.ShapeDtypeStruct((T, EMB), jnp.float32),
    )(h01, h01, gts, W_g)


def kernel(x, W_router, W_h, W_g, k_ffwd, v_ffwd):
    B, Tn, D = x.shape
    x2d = x.reshape(Tn, D)
    qi, gts, dest, meta = _router_proj(x2d, W_router, W_h)
    dest_flat = dest.reshape(NA)
    tok_ids = jnp.arange(NA, dtype=jnp.int32) % Tn
    packed_q = _sc_pack(dest_flat, tok_ids, qi)
    be_arr = meta[0, :MAXB]
    nbt_arr = meta[1, :1]
    h_packed = _ffn_block(be_arr, nbt_arr, packed_q, k_ffwd, v_ffwd)
    h01 = _sc_unpack(dest_flat, h_packed)
    out = _combine(h01, gts, W_g)
    return out.reshape(B, Tn, D)


# speedup vs baseline: 1.1396x; 1.1396x over previous
import functools
import math

import jax
import jax.numpy as jnp
from jax import lax
from jax.experimental import pallas as pl
from jax.experimental.pallas import tpu as pltpu
from jax.experimental.pallas import tpu_sc as plsc

EMB = 1024
H = 16
HD = 64
E = 8
S = 512
T = 2048
TB = 256
PB = 256
MAXB = T // PB * (E // 2) + E
PMAX = MAXB * PB
NA = 2 * T
EMB2 = EMB // 2

_INV_SQRT2 = 1.0 / math.sqrt(2.0)


def _gelu(s):
    return 0.5 * s * (1.0 + lax.erf(s * _INV_SQRT2))


def _pack_row(y):
    bl = lax.bitcast_convert_type(y[:, :EMB2], jnp.int32)
    br = lax.bitcast_convert_type(y[:, EMB2:], jnp.int32)
    hi = (bl + 0x8000) & jnp.int32(-65536)
    lo = lax.shift_right_logical(br + 0x8000, 16)
    return hi | lo


def _unpack_row(w):
    yl = lax.bitcast_convert_type(w & jnp.int32(-65536), jnp.float32)
    yr = lax.bitcast_convert_type(lax.shift_left(w, 16), jnp.float32)
    return jnp.concatenate([yl, yr], axis=1)


def _router_proj_kernel(x_ref, wr_ref, wh_ref, q_ref, gts_ref, dest_ref,
                        meta_ref, aidx_ref):
    b = pl.program_id(0)
    xb = x_ref[...]
    logits = lax.dot_general(wr_ref[...], xb, (((1,), (1,)), ((), ())),
                             preferred_element_type=jnp.float32)
    e_iota = lax.broadcasted_iota(jnp.int32, (E, TB), 0)
    m1 = jnp.max(logits, axis=0, keepdims=True)
    a1 = jnp.min(jnp.where(logits == m1, e_iota, E), axis=0, keepdims=True)
    l2 = jnp.where(e_iota == a1, -jnp.inf, logits)
    m2 = jnp.max(l2, axis=0, keepdims=True)
    a2 = jnp.min(jnp.where(l2 == m2, e_iota, E), axis=0, keepdims=True)
    g1 = 0.5 * jax.nn.sigmoid(m1)
    g2 = 0.5 * jax.nn.sigmoid(m2)
    aidx_ref[:, :, pl.ds(b * TB, TB)] = (
        jnp.concatenate([a1, a2], axis=0).reshape(2, 1, TB))
    gts_ref[...] = jnp.concatenate([g1, g2], axis=0).reshape(2, 1, TB)
    q = lax.dot_general(xb, wh_ref[...], (((1,), (1,)), ((), ())),
                        preferred_element_type=jnp.float32)
    q_ref[...] = _pack_row(q)

    @pl.when(b == T // TB - 1)
    def _():
        _meta_body(aidx_ref, dest_ref, meta_ref)


def _router_proj(x2d, W_router, W_h):
    return pl.pallas_call(
        _router_proj_kernel,
        grid=(T // TB,),
        in_specs=[
            pl.BlockSpec((TB, EMB), lambda b: (b, 0)),
            pl.BlockSpec((E, EMB), lambda b: (0, 0)),
            pl.BlockSpec((EMB, EMB), lambda b: (0, 0)),
        ],
        out_specs=[
            pl.BlockSpec((TB, EMB2), lambda b: (b, 0)),
            pl.BlockSpec((2, 1, TB), lambda b: (0, 0, b)),
            pl.BlockSpec((2, 1, T), lambda b: (0, 0, 0)),
            pl.BlockSpec((2, 128), lambda b: (0, 0)),
        ],
        out_shape=[
            jax.ShapeDtypeStruct((T, EMB2), jnp.int32),
            jax.ShapeDtypeStruct((2, 1, T), jnp.float32),
            jax.ShapeDtypeStruct((2, 1, T), jnp.int32),
            jax.ShapeDtypeStruct((2, 128), jnp.int32),
        ],
        scratch_shapes=[pltpu.VMEM((2, 1, T), jnp.int32)],
    )(x2d, W_router, W_h)


def _meta_body(aidx_ref, dest_ref, meta_ref):
    a1 = aidx_ref[0]
    a2 = aidx_ref[1]
    e_iota = lax.broadcasted_iota(jnp.int32, (E, T), 0)
    occ1 = (e_iota == a1).astype(jnp.int32)
    occ2 = (e_iota == a2).astype(jnp.int32)
    occ = occ1 + occ2
    c = occ
    sh = 1
    while sh < T:
        c = c + jnp.concatenate([jnp.zeros((E, sh), jnp.int32), c[:, :T - sh]], axis=1)
        sh *= 2
    rank = c - occ
    count = c[:, T - 1:T]
    pc = ((count + PB - 1) // PB) * PB
    o = pc
    sh = 1
    while sh < E:
        o = o + jnp.concatenate([jnp.zeros((sh, 1), jnp.int32), o[:E - sh]], axis=0)
        sh *= 2
    excl = o - pc
    nb_e = pc // PB
    blk_off = excl // PB
    nb_total = jnp.sum(nb_e, axis=0, keepdims=True)
    e_col = lax.broadcasted_iota(jnp.int32, (E, 128), 0)
    b_iota = lax.broadcasted_iota(jnp.int32, (E, 128), 1)
    belong = (b_iota >= blk_off) & (b_iota < blk_off + nb_e)
    be = jnp.sum(jnp.where(belong, e_col, 0), axis=0, keepdims=True)
    last = nb_total - 1
    bl = (blk_off <= last) & (last < blk_off + nb_e)
    e_last = jnp.sum(jnp.where(bl, e_col[:, :1], 0), axis=0, keepdims=True)
    b_row = lax.broadcasted_iota(jnp.int32, (1, 128), 1)
    be_final = jnp.where(b_row < nb_total, be, e_last)
    meta_ref[...] = jnp.concatenate(
        [be_final, jnp.broadcast_to(nb_total, (1, 128))], axis=0)
    slot = excl + rank
    d1 = jnp.sum(occ1 * slot, axis=0, keepdims=True)
    d2 = jnp.sum(occ2 * slot, axis=0, keepdims=True)
    dest_ref[...] = jnp.concatenate([d1, d2], axis=0).reshape(2, 1, T)


_SC_CHUNK = 128


def _sc_mesh():
    return plsc.VectorSubcoreMesh(core_axis_name="c", subcore_axis_name="s")


def _sc_pack(dest_flat, tok_ids, qi):
    info = plsc.get_sparse_core_info()
    nw = info.num_cores * info.num_subcores
    per_w = NA // nw

    @functools.partial(
        pl.kernel,
        mesh=_sc_mesh(),
        out_type=jax.ShapeDtypeStruct((PMAX, EMB2), jnp.int32),
        scratch_types=[
            pltpu.VMEM((_SC_CHUNK,), jnp.int32),
            pltpu.VMEM((_SC_CHUNK,), jnp.int32),
            pltpu.VMEM((_SC_CHUNK, EMB2), jnp.int32),
            pltpu.SemaphoreType.DMA,
        ],
    )
    def k(dest_hbm, tok_hbm, q_hbm, pq_hbm, tidx_v, didx_v, rows_v, sem):
        wid = lax.axis_index("s") * info.num_cores + lax.axis_index("c")
        base = wid * per_w
        for c in range(per_w // _SC_CHUNK):
            off = base + c * _SC_CHUNK
            pltpu.sync_copy(tok_hbm.at[pl.ds(off, _SC_CHUNK)], tidx_v)
            pltpu.sync_copy(dest_hbm.at[pl.ds(off, _SC_CHUNK)], didx_v)
            pltpu.async_copy(q_hbm.at[tidx_v], rows_v, sem).wait()
            pltpu.async_copy(rows_v, pq_hbm.at[didx_v], sem).wait()

    return k(dest_flat, tok_ids, qi)


def _sc_unpack(dest_flat, hpi):
    info = plsc.get_sparse_core_info()
    nw = info.num_cores * info.num_subcores
    per_w = NA // nw

    @functools.partial(
        pl.kernel,
        mesh=_sc_mesh(),
        out_type=jax.ShapeDtypeStruct((NA, EMB2), jnp.int32),
        scratch_types=[
            pltpu.VMEM((_SC_CHUNK,), jnp.int32),
            pltpu.VMEM((_SC_CHUNK, EMB2), jnp.int32),
            pltpu.SemaphoreType.DMA,
        ],
    )
    def k(dest_hbm, hp_hbm, h01_hbm, didx_v, rows_v, sem):
        wid = lax.axis_index("s") * info.num_cores + lax.axis_index("c")
        base = wid * per_w
        for c in range(per_w // _SC_CHUNK):
            off = base + c * _SC_CHUNK
            pltpu.sync_copy(dest_hbm.at[pl.ds(off, _SC_CHUNK)], didx_v)
            pltpu.async_copy(hp_hbm.at[didx_v], rows_v, sem).wait()
            pltpu.sync_copy(rows_v, h01_hbm.at[pl.ds(off, _SC_CHUNK)])

    return k(dest_flat, hpi)


def _ffn_block_kernel(be_ref, nbt_ref, pq_ref, k_ref, v_ref, hp_ref):
    b = pl.program_id(0)

    @pl.when(b < nbt_ref[0])
    def _():
        qb = _unpack_row(pq_ref[...])
        hcols = []
        for h in range(H):
            qh = qb[:, h * HD:(h + 1) * HD]
            s = lax.dot_general(qh, k_ref[h, 0], (((1,), (1,)), ((), ())),
                                preferred_element_type=jnp.float32)
            a = s + s * lax.erf(s * _INV_SQRT2)
            hcols.append(lax.dot_general(a, v_ref[h, 0], (((1,), (0,)), ((), ())),
                                         preferred_element_type=jnp.float32))
        hp_ref[...] = _pack_row(jnp.concatenate(hcols, axis=1))


def _ffn_block(be_arr, nbt_arr, packed_q, k_ffwd, v_ffwd):
    grid_spec = pltpu.PrefetchScalarGridSpec(
        num_scalar_prefetch=2,
        grid=(MAXB,),
        in_specs=[
            pl.BlockSpec((PB, EMB2), lambda b, be, nbt: (b, 0)),
            pl.BlockSpec((H, 1, S, HD), lambda b, be, nbt: (0, be[b], 0, 0)),
            pl.BlockSpec((H, 1, S, HD), lambda b, be, nbt: (0, be[b], 0, 0)),
        ],
        out_specs=pl.BlockSpec((PB, EMB2), lambda b, be, nbt: (b, 0)),
    )
    return pl.pallas_call(
        _ffn_block_kernel,
        grid_spec=grid_spec,
        out_shape=jax.ShapeDtypeStruct((PMAX, EMB2), jnp.int32),
    )(be_arr, nbt_arr, packed_q, k_ffwd, v_ffwd)


def _combine_kernel(h0_ref, h1_ref, gts_ref, wg_ref, o_ref):
    g0 = gts_ref[0, 0, :].reshape(TB, 1)
    g1 = gts_ref[1, 0, :].reshape(TB, 1)
    mixed = g0 * _unpack_row(h0_ref[...]) + g1 * _unpack_row(h1_ref[...])
    o_ref[...] = lax.dot_general(mixed, wg_ref[...], (((1,), (1,)), ((), ())),
                                 preferred_element_type=jnp.float32)


def _combine(h01, gts, W_g):
    nblk = T // TB
    return pl.pallas_call(
        _combine_kernel,
        grid=(nblk,),
        in_specs=[
            pl.BlockSpec((TB, EMB2), lambda b: (b, 0)),
            pl.BlockSpec((TB, EMB2), lambda b: (b + nblk, 0)),
            pl.BlockSpec((2, 1, TB), lambda b: (0, 0, b)),
            pl.BlockSpec((EMB, EMB), lambda b: (0, 0)),
        ],
        out_specs=pl.BlockSpec((TB, EMB), lambda b: (b, 0)),
        out_shape=jax.ShapeDtypeStruct((T, EMB), jnp.float32),
    )(h01, h01, gts, W_g)


def kernel(x, W_router, W_h, W_g, k_ffwd, v_ffwd):
    B, Tn, D = x.shape
    x2d = x.reshape(Tn, D)
    qi, gts, dest, meta = _router_proj(x2d, W_router, W_h)
    dest_flat = dest.reshape(NA)
    tok_ids = jnp.arange(NA, dtype=jnp.int32) % Tn
    packed_q = _sc_pack(dest_flat, tok_ids, qi)
    be_arr = meta[0, :MAXB]
    nbt_arr = meta[1, :1]
    h_packed = _ffn_block(be_arr, nbt_arr, packed_q, k_ffwd, v_ffwd)
    h01 = _sc_unpack(dest_flat, h_packed)
    out = _combine(h01, gts, W_g)
    return out.reshape(B, Tn, D)


# speedup vs baseline: 1.2237x; 1.0739x over previous
import functools
import math

import jax
import jax.numpy as jnp
from jax import lax
from jax.experimental import pallas as pl
from jax.experimental.pallas import tpu as pltpu
from jax.experimental.pallas import tpu_sc as plsc

EMB = 1024
H = 16
HD = 64
E = 8
S = 512
T = 2048
TB = 256
PB = 256
MAXB = T // PB * (E // 2) + E
PMAX = MAXB * PB
NA = 2 * T
EMB2 = EMB // 2

_INV_SQRT2 = 1.0 / math.sqrt(2.0)


def _gelu(s):
    return 0.5 * s * (1.0 + lax.erf(s * _INV_SQRT2))


def _pack_row(y):
    bl = lax.bitcast_convert_type(y[:, :EMB2], jnp.int32)
    br = lax.bitcast_convert_type(y[:, EMB2:], jnp.int32)
    hi = (bl + 0x8000) & jnp.int32(-65536)
    lo = lax.shift_right_logical(br + 0x8000, 16)
    return hi | lo


def _unpack_row(w):
    yl = lax.bitcast_convert_type(w & jnp.int32(-65536), jnp.float32)
    yr = lax.bitcast_convert_type(lax.shift_left(w, 16), jnp.float32)
    return jnp.concatenate([yl, yr], axis=1)


def _router_proj_kernel(x_ref, wr_ref, wh_ref, q_ref, gts_ref, dest_ref,
                        meta_ref, aidx_ref):
    b = pl.program_id(0)
    xb = x_ref[...]
    logits = lax.dot_general(wr_ref[...], xb, (((1,), (1,)), ((), ())),
                             preferred_element_type=jnp.float32)
    e_iota = lax.broadcasted_iota(jnp.int32, (E, TB), 0)
    m1 = jnp.max(logits, axis=0, keepdims=True)
    a1 = jnp.min(jnp.where(logits == m1, e_iota, E), axis=0, keepdims=True)
    l2 = jnp.where(e_iota == a1, -jnp.inf, logits)
    m2 = jnp.max(l2, axis=0, keepdims=True)
    a2 = jnp.min(jnp.where(l2 == m2, e_iota, E), axis=0, keepdims=True)
    g1 = 0.5 * jax.nn.sigmoid(m1)
    g2 = 0.5 * jax.nn.sigmoid(m2)
    aidx_ref[:, :, pl.ds(b * TB, TB)] = (
        jnp.concatenate([a1, a2], axis=0).reshape(2, 1, TB))
    gts_ref[...] = jnp.concatenate([g1, g2], axis=0).reshape(2, 1, TB)
    q = lax.dot_general(xb, wh_ref[...], (((1,), (1,)), ((), ())),
                        preferred_element_type=jnp.float32)
    q_ref[...] = _pack_row(q)

    @pl.when(b == T // TB - 1)
    def _():
        _meta_body(aidx_ref, dest_ref, meta_ref)


def _router_proj(x2d, W_router, W_h):
    return pl.pallas_call(
        _router_proj_kernel,
        grid=(T // TB,),
        in_specs=[
            pl.BlockSpec((TB, EMB), lambda b: (b, 0)),
            pl.BlockSpec((E, EMB), lambda b: (0, 0)),
            pl.BlockSpec((EMB, EMB), lambda b: (0, 0)),
        ],
        out_specs=[
            pl.BlockSpec((TB, EMB2), lambda b: (b, 0)),
            pl.BlockSpec((2, 1, TB), lambda b: (0, 0, b)),
            pl.BlockSpec((2, 1, T), lambda b: (0, 0, 0)),
            pl.BlockSpec((2, 128), lambda b: (0, 0)),
        ],
        out_shape=[
            jax.ShapeDtypeStruct((T, EMB2), jnp.int32),
            jax.ShapeDtypeStruct((2, 1, T), jnp.float32),
            jax.ShapeDtypeStruct((2, 1, T), jnp.int32),
            jax.ShapeDtypeStruct((2, 128), jnp.int32),
        ],
        scratch_shapes=[pltpu.VMEM((2, 1, T), jnp.int32)],
    )(x2d, W_router, W_h)


def _meta_body(aidx_ref, dest_ref, meta_ref):
    a1 = aidx_ref[0]
    a2 = aidx_ref[1]
    e_iota = lax.broadcasted_iota(jnp.int32, (E, T), 0)
    occ1 = (e_iota == a1).astype(jnp.int32)
    occ2 = (e_iota == a2).astype(jnp.int32)
    occ = occ1 + occ2
    c = occ
    sh = 1
    while sh < T:
        c = c + jnp.concatenate([jnp.zeros((E, sh), jnp.int32), c[:, :T - sh]], axis=1)
        sh *= 2
    rank = c - occ
    count = c[:, T - 1:T]
    pc = ((count + PB - 1) // PB) * PB
    o = pc
    sh = 1
    while sh < E:
        o = o + jnp.concatenate([jnp.zeros((sh, 1), jnp.int32), o[:E - sh]], axis=0)
        sh *= 2
    excl = o - pc
    nb_e = pc // PB
    blk_off = excl // PB
    nb_total = jnp.sum(nb_e, axis=0, keepdims=True)
    e_col = lax.broadcasted_iota(jnp.int32, (E, 128), 0)
    b_iota = lax.broadcasted_iota(jnp.int32, (E, 128), 1)
    belong = (b_iota >= blk_off) & (b_iota < blk_off + nb_e)
    be = jnp.sum(jnp.where(belong, e_col, 0), axis=0, keepdims=True)
    last = nb_total - 1
    bl = (blk_off <= last) & (last < blk_off + nb_e)
    e_last = jnp.sum(jnp.where(bl, e_col[:, :1], 0), axis=0, keepdims=True)
    b_row = lax.broadcasted_iota(jnp.int32, (1, 128), 1)
    be_final = jnp.where(b_row < nb_total, be, e_last)
    meta_ref[...] = jnp.concatenate(
        [be_final, jnp.broadcast_to(nb_total, (1, 128))], axis=0)
    slot = excl + rank
    d1 = jnp.sum(occ1 * slot, axis=0, keepdims=True)
    d2 = jnp.sum(occ2 * slot, axis=0, keepdims=True)
    dest_ref[...] = jnp.concatenate([d1, d2], axis=0).reshape(2, 1, T)


_SC_CHUNK = 128


def _sc_mesh():
    return plsc.VectorSubcoreMesh(core_axis_name="c", subcore_axis_name="s")


def _sc_pack(dest_flat, tok_ids, qi):
    info = plsc.get_sparse_core_info()
    nw = info.num_cores * info.num_subcores
    per_w = NA // nw

    @functools.partial(
        pl.kernel,
        mesh=_sc_mesh(),
        out_type=jax.ShapeDtypeStruct((PMAX, EMB2), jnp.int32),
        scratch_types=[
            pltpu.VMEM((_SC_CHUNK,), jnp.int32),
            pltpu.VMEM((_SC_CHUNK,), jnp.int32),
            pltpu.VMEM((_SC_CHUNK, EMB2), jnp.int32),
            pltpu.SemaphoreType.DMA,
        ],
    )
    def k(dest_hbm, tok_hbm, q_hbm, pq_hbm, tidx_v, didx_v, rows_v, sem):
        wid = lax.axis_index("s") * info.num_cores + lax.axis_index("c")
        base = wid * per_w
        for c in range(per_w // _SC_CHUNK):
            off = base + c * _SC_CHUNK
            pltpu.sync_copy(tok_hbm.at[pl.ds(off, _SC_CHUNK)], tidx_v)
            pltpu.sync_copy(dest_hbm.at[pl.ds(off, _SC_CHUNK)], didx_v)
            pltpu.async_copy(q_hbm.at[tidx_v], rows_v, sem).wait()
            pltpu.async_copy(rows_v, pq_hbm.at[didx_v], sem).wait()

    return k(dest_flat, tok_ids, qi)


def _sc_unpack(dest_flat, hpi):
    info = plsc.get_sparse_core_info()
    nw = info.num_cores * info.num_subcores
    per_w = NA // nw

    @functools.partial(
        pl.kernel,
        mesh=_sc_mesh(),
        out_type=jax.ShapeDtypeStruct((NA, EMB2), jnp.int32),
        scratch_types=[
            pltpu.VMEM((_SC_CHUNK,), jnp.int32),
            pltpu.VMEM((_SC_CHUNK, EMB2), jnp.int32),
            pltpu.SemaphoreType.DMA,
        ],
    )
    def k(dest_hbm, hp_hbm, h01_hbm, didx_v, rows_v, sem):
        wid = lax.axis_index("s") * info.num_cores + lax.axis_index("c")
        base = wid * per_w
        for c in range(per_w // _SC_CHUNK):
            off = base + c * _SC_CHUNK
            pltpu.sync_copy(dest_hbm.at[pl.ds(off, _SC_CHUNK)], didx_v)
            pltpu.async_copy(hp_hbm.at[didx_v], rows_v, sem).wait()
            pltpu.sync_copy(rows_v, h01_hbm.at[pl.ds(off, _SC_CHUNK)])

    return k(dest_flat, hpi)


def _ffn_block_kernel(be_ref, nbt_ref, pq_ref, k_ref, v_ref, hp_ref):
    b = pl.program_id(0)

    @pl.when(b < nbt_ref[0])
    def _():
        qb = _unpack_row(pq_ref[...])
        hcols = []
        for h in range(H):
            qh = qb[:, h * HD:(h + 1) * HD]
            s = lax.dot_general(qh, k_ref[h, 0], (((1,), (1,)), ((), ())),
                                preferred_element_type=jnp.float32)
            a = s + s * lax.erf(s * _INV_SQRT2)
            hcols.append(lax.dot_general(a, v_ref[h, 0], (((1,), (0,)), ((), ())),
                                         preferred_element_type=jnp.float32))
        hp_ref[...] = _pack_row(jnp.concatenate(hcols, axis=1))


def _ffn_block(be_arr, nbt_arr, packed_q, k_ffwd, v_ffwd):
    grid_spec = pltpu.PrefetchScalarGridSpec(
        num_scalar_prefetch=2,
        grid=(MAXB,),
        in_specs=[
            pl.BlockSpec((PB, EMB2),
                         lambda b, be, nbt: (jnp.minimum(b, nbt[0] - 1), 0)),
            pl.BlockSpec((H, 1, S, HD), lambda b, be, nbt: (0, be[b], 0, 0)),
            pl.BlockSpec((H, 1, S, HD), lambda b, be, nbt: (0, be[b], 0, 0)),
        ],
        out_specs=pl.BlockSpec((PB, EMB2),
                               lambda b, be, nbt: (jnp.minimum(b, nbt[0] - 1), 0)),
    )
    return pl.pallas_call(
        _ffn_block_kernel,
        grid_spec=grid_spec,
        out_shape=jax.ShapeDtypeStruct((PMAX, EMB2), jnp.int32),
    )(be_arr, nbt_arr, packed_q, k_ffwd, v_ffwd)


def _combine_kernel(h0_ref, h1_ref, gts_ref, wg_ref, o_ref):
    g0 = gts_ref[0, 0, :].reshape(TB, 1)
    g1 = gts_ref[1, 0, :].reshape(TB, 1)
    mixed = g0 * _unpack_row(h0_ref[...]) + g1 * _unpack_row(h1_ref[...])
    o_ref[...] = lax.dot_general(mixed, wg_ref[...], (((1,), (1,)), ((), ())),
                                 preferred_element_type=jnp.float32)


def _combine(h01, gts, W_g):
    nblk = T // TB
    return pl.pallas_call(
        _combine_kernel,
        grid=(nblk,),
        in_specs=[
            pl.BlockSpec((TB, EMB2), lambda b: (b, 0)),
            pl.BlockSpec((TB, EMB2), lambda b: (b + nblk, 0)),
            pl.BlockSpec((2, 1, TB), lambda b: (0, 0, b)),
            pl.BlockSpec((EMB, EMB), lambda b: (0, 0)),
        ],
        out_specs=pl.BlockSpec((TB, EMB), lambda b: (b, 0)),
        out_shape=jax.ShapeDtypeStruct((T, EMB), jnp.float32),
    )(h01, h01, gts, W_g)


def kernel(x, W_router, W_h, W_g, k_ffwd, v_ffwd):
    B, Tn, D = x.shape
    x2d = x.reshape(Tn, D)
    qi, gts, dest, meta = _router_proj(x2d, W_router, W_h)
    dest_flat = dest.reshape(NA)
    tok_ids = jnp.arange(NA, dtype=jnp.int32) % Tn
    packed_q = _sc_pack(dest_flat, tok_ids, qi)
    be_arr = meta[0, :MAXB]
    nbt_arr = meta[1, :1]
    h_packed = _ffn_block(be_arr, nbt_arr, packed_q, k_ffwd, v_ffwd)
    h01 = _sc_unpack(dest_flat, h_packed)
    out = _combine(h01, gts, W_g)
    return out.reshape(B, Tn, D)
